# baseline (speedup 1.0000x reference)
import functools

import jax
import jax.numpy as jnp
from jax import lax
from jax.experimental import pallas as pl
from jax.experimental.pallas import tpu as pltpu
from jax.experimental.pallas import tpu_sc as plsc

N = 10000
E = 320000
C = 128
K = 27
EPS = 1e-5

NUM_CORES = 2
NUM_SUBCORES = 16
NW = NUM_CORES * NUM_SUBCORES
EPW = E // NW
CHUNK = 80
NCH = EPW // CHUNK
BCH = 25
NBLK_E = NCH // BCH
ZB = 80
NZB = N // ZB

BN_ROWS = 1000
NBLK = N // BN_ROWS
MM_ROWS = 5000
NBLK_MM = N // MM_ROWS
ST_ROWS = 2000
NBLK_ST = N // ST_ROWS


def _xw_matmul(x, W):
    def body(x_ref, w_ref, o_ref):
        o_ref[...] = jnp.dot(x_ref[...], w_ref[0],
                             preferred_element_type=jnp.float32
)

    return pl.pallas_call(
        body,
        grid=(NBLK_MM, K),
        in_specs=[
            pl.BlockSpec((MM_ROWS, C), lambda b, k: (b, 0)),
            pl.BlockSpec((1, C, C), lambda b, k: (k, 0, 0)),
        ],
        out_specs=pl.BlockSpec((MM_ROWS, C),
                               lambda b, k: (k * NBLK_MM + b, 0)),
        out_shape=jax.ShapeDtypeStruct((K * N, C), jnp.float32),
    )(x, W)


def _sc_edge_scatter(xw, er, kidr):
    mesh = plsc.VectorSubcoreMesh(core_axis_name="c", subcore_axis_name="s")

    @functools.partial(
        pl.kernel,
        mesh=mesh,
        out_type=jax.ShapeDtypeStruct((NUM_CORES, N, C), jnp.float32),
        scratch_types=[
            pltpu.VMEM((BCH, CHUNK), jnp.int32),
            pltpu.VMEM((BCH, CHUNK), jnp.int32),
            pltpu.VMEM((BCH, CHUNK), jnp.int32),
            pltpu.VMEM((BCH, CHUNK), jnp.int32),
            pltpu.VMEM((BCH, CHUNK), jnp.int32),
            pltpu.VMEM((CHUNK, C), jnp.float32),
            pltpu.VMEM((CHUNK, C), jnp.float32),
            pltpu.VMEM_SHARED((N, C), jnp.float32),
            pltpu.SemaphoreType.DMA,
            pltpu.SemaphoreType.DMA,
            pltpu.SemaphoreType.DMA,
            pltpu.SemaphoreType.DMA,
            pltpu.SemaphoreType.DMA,
        ],
    )
    def k(er_hbm, kid_hbm, xw_hbm, out_hbm,
          p2a, p2b, d2a, d2b, kid1, rows_a, rows_b, acc,
          sem_a, sem_b, sem_ma, sem_mb, sem_k):
        cid = lax.axis_index("c")
        sid = lax.axis_index("s")
        wid = sid * NUM_CORES + cid

        pltpu.make_async_copy(er_hbm.at[0, wid, 0], p2a, sem_ma).start()
        pltpu.make_async_copy(er_hbm.at[1, wid, 0], d2a, sem_ma).start()
        pltpu.make_async_copy(kid_hbm.at[wid, 0], kid1, sem_k).start()

        def zrow(i, carry):
            for j in range(C // 16):
                rows_a[i, pl.ds(j * 16, 16)] = jnp.zeros((16,), jnp.float32)
            return carry
        lax.fori_loop(0, CHUNK, zrow, 0)
        for t in range(NZB // NUM_SUBCORES + 1):
            b = sid + t * NUM_SUBCORES

            @pl.when(b < NZB)
            def _():
                pltpu.make_async_copy(rows_a, acc.at[pl.ds(b * ZB, ZB)],
                                      sem_a).start()
        for t in range(NZB // NUM_SUBCORES + 1):
            b = sid + t * NUM_SUBCORES

            @pl.when(b < NZB)
            def _():
                pltpu.make_async_copy(rows_a, acc.at[pl.ds(b * ZB, ZB)],
                                      sem_a).wait()

        plsc.subcore_barrier()

        for bi in range(NBLK_E):
            if bi % 2 == 0:
                pb, db, sm = p2a, d2a, sem_ma
                pn, dn, smn = p2b, d2b, sem_mb
            else:
                pb, db, sm = p2b, d2b, sem_mb
                pn, dn, smn = p2a, d2a, sem_ma

            pltpu.make_async_copy(er_hbm.at[0, wid, bi], pb, sm).wait()
            pltpu.make_async_copy(er_hbm.at[1, wid, bi], db, sm).wait()
            pltpu.make_async_copy(kid_hbm.at[wid, bi], kid1, sem_k).wait()
            if bi + 1 < NBLK_E:
                pltpu.make_async_copy(er_hbm.at[0, wid, bi + 1], pn,
                                      smn).start()
                pltpu.make_async_copy(er_hbm.at[1, wid, bi + 1], dn,
                                      smn).start()

            def gidx(i, carry):
                for j in range(CHUNK // 16):
                    sl = pl.ds(j * 16, 16)
                    pb[i, sl] = kid1[i, sl] * N + pb[i, sl]
                return carry
            lax.fori_loop(0, BCH, gidx, 0)

            if bi + 1 < NBLK_E:
                pltpu.make_async_copy(kid_hbm.at[wid, bi + 1], kid1,
                                      sem_k).start()

            def gather(ci, rows, sem, pb=pb):
                return pltpu.make_async_copy(xw_hbm.at[pb.at[ci]], rows, sem)

            gather(0, rows_a, sem_a).start()

            def step(t, carry, gather=gather, db=db):
                c = 2 * t
                gather(c + 1, rows_b, sem_b).start()
                gather(c, rows_a, sem_a).wait()
                pltpu.sync_copy(rows_a, acc.at[db.at[c]], add=True)
                gather(c + 2, rows_a, sem_a).start()
                gather(c + 1, rows_b, sem_b).wait()
                pltpu.sync_copy(rows_b, acc.at[db.at[c + 1]], add=True)
                return carry
            lax.fori_loop(0, (BCH - 1) // 2, step, 0)

            gather(BCH - 1, rows_a, sem_a).wait()
            pltpu.sync_copy(rows_a, acc.at[db.at[BCH - 1]], add=True)

        plsc.subcore_barrier()

        for t in range(NZB // NUM_SUBCORES + 1):
            b = sid + t * NUM_SUBCORES

            @pl.when(b < NZB)
            def _():
                pltpu.make_async_copy(acc.at[pl.ds(b * ZB, ZB)],
                                      out_hbm.at[cid, pl.ds(b * ZB, ZB)],
                                      sem_a).start()
        for t in range(NZB // NUM_SUBCORES + 1):
            b = sid + t * NUM_SUBCORES

            @pl.when(b < NZB)
            def _():
                pltpu.make_async_copy(acc.at[pl.ds(b * ZB, ZB)],
                                      out_hbm.at[cid, pl.ds(b * ZB, ZB)],
                                      sem_a).wait()

    return k(er, kidr, xw)


def _bn_relu(partial, gamma2, beta2):
    def body(p_ref, g_ref, b_ref, o_ref, s_vmem, st_vmem):
        p = pl.program_id(0)
        b = pl.program_id(1)

        @pl.when(p == 0)
        def _():
            s = p_ref[0] + p_ref[1]
            s_vmem[pl.ds(b * ST_ROWS, ST_ROWS), :] = s

            @pl.when(b == 0)
            def _():
                st_vmem[...] = jnp.zeros_like(st_vmem)

            st_vmem[0, :] += jnp.sum(s, axis=0)
            st_vmem[1, :] += jnp.sum(s * s, axis=0)
            o_ref[...] = s

        @pl.when(p == 1)
        def _():
            mean = st_vmem[0:1, :] * (1.0 / N)
            var = st_vmem[1:2, :] * (1.0 / N) - mean * mean
            scale = g_ref[...] * lax.rsqrt(var + EPS)
            s = s_vmem[pl.ds(b * ST_ROWS, ST_ROWS), :]
            o_ref[...] = jnp.maximum((s - mean) * scale + b_ref[...], 0.0)

    return pl.pallas_call(
        body,
        grid=(2, NBLK_ST),
        in_specs=[
            pl.BlockSpec((NUM_CORES, ST_ROWS, C),
                         lambda p, b: (0, b * (1 - p), 0)),
            pl.BlockSpec((1, C), lambda p, b: (0, 0)),
            pl.BlockSpec((1, C), lambda p, b: (0, 0)),
        ],
        out_specs=pl.BlockSpec((ST_ROWS, C), lambda p, b: (b, 0)),
        out_shape=jax.ShapeDtypeStruct((N, C), jnp.float32),
        scratch_shapes=[
            pltpu.VMEM((N, C), jnp.float32),
            pltpu.VMEM((8, C), jnp.float32),
        ],
    )(partial, gamma2, beta2)


def kernel(x, edge_index, kernel_id, W, gamma, beta):
    er = edge_index.astype(jnp.int32).reshape(2, NW, NBLK_E, BCH, CHUNK)
    kidr = kernel_id.astype(jnp.int32).reshape(NW, NBLK_E, BCH, CHUNK)

    xw = _xw_matmul(x, W)
    partial = _sc_edge_scatter(xw, er, kidr)
    return _bn_relu(partial, gamma.reshape(1, C), beta.reshape(1, C))


# speedup vs baseline: 1.0049x; 1.0049x over previous
import functools

import jax
import jax.numpy as jnp
from jax import lax
from jax.experimental import pallas as pl
from jax.experimental.pallas import tpu as pltpu
from jax.experimental.pallas import tpu_sc as plsc

N = 10000
E = 320000
C = 128
K = 27
EPS = 1e-5

NUM_CORES = 2
NUM_SUBCORES = 16
NW = NUM_CORES * NUM_SUBCORES
EPW = E // NW
CHUNK = 80
NCH = EPW // CHUNK
BCH = 25
NBLK_E = NCH // BCH
ZB = 80
NZB = N // ZB

BN_ROWS = 1000
NBLK = N // BN_ROWS
MM_ROWS = 5000
NBLK_MM = N // MM_ROWS
ST_ROWS = 2000
NBLK_ST = N // ST_ROWS


def _xw_matmul(x, W):
    def body(x_ref, w_ref, o_ref):
        o_ref[...] = jnp.dot(x_ref[...], w_ref[0],
                             preferred_element_type=jnp.float32
)

    return pl.pallas_call(
        body,
        grid=(NBLK_MM, K),
        in_specs=[
            pl.BlockSpec((MM_ROWS, C), lambda b, k: (b, 0)),
            pl.BlockSpec((1, C, C), lambda b, k: (k, 0, 0)),
        ],
        out_specs=pl.BlockSpec((MM_ROWS, C),
                               lambda b, k: (k * NBLK_MM + b, 0)),
        out_shape=jax.ShapeDtypeStruct((K * N, C), jnp.float32),
    )(x, W)


def _sc_edge_scatter(xw, er, kidr):
    mesh = plsc.VectorSubcoreMesh(core_axis_name="c", subcore_axis_name="s")

    @functools.partial(
        pl.kernel,
        mesh=mesh,
        out_type=jax.ShapeDtypeStruct((NUM_CORES, N, C), jnp.float32),
        scratch_types=[
            pltpu.VMEM((BCH, CHUNK), jnp.int32),
            pltpu.VMEM((BCH, CHUNK), jnp.int32),
            pltpu.VMEM((BCH, CHUNK), jnp.int32),
            pltpu.VMEM((BCH, CHUNK), jnp.int32),
            pltpu.VMEM((BCH, CHUNK), jnp.int32),
            pltpu.VMEM((CHUNK, C), jnp.float32),
            pltpu.VMEM((CHUNK, C), jnp.float32),
            pltpu.VMEM_SHARED((N, C), jnp.float32),
            pltpu.SemaphoreType.DMA,
            pltpu.SemaphoreType.DMA,
            pltpu.SemaphoreType.DMA,
            pltpu.SemaphoreType.DMA,
            pltpu.SemaphoreType.DMA,
        ],
    )
    def k(er_hbm, kid_hbm, xw_hbm, out_hbm,
          p2a, p2b, d2a, d2b, kid1, rows_a, rows_b, acc,
          sem_a, sem_b, sem_ma, sem_mb, sem_k):
        cid = lax.axis_index("c")
        sid = lax.axis_index("s")
        wid = sid * NUM_CORES + cid

        pltpu.make_async_copy(er_hbm.at[0, wid, 0], p2a, sem_ma).start()
        pltpu.make_async_copy(er_hbm.at[1, wid, 0], d2a, sem_ma).start()
        pltpu.make_async_copy(kid_hbm.at[wid, 0], kid1, sem_k).start()

        def zrow(i, carry):
            for j in range(C // 16):
                rows_a[i, pl.ds(j * 16, 16)] = jnp.zeros((16,), jnp.float32)
            return carry
        lax.fori_loop(0, CHUNK, zrow, 0)
        for t in range(NZB // NUM_SUBCORES + 1):
            b = sid + t * NUM_SUBCORES

            @pl.when(b < NZB)
            def _():
                pltpu.make_async_copy(rows_a, acc.at[pl.ds(b * ZB, ZB)],
                                      sem_a).start()
        for t in range(NZB // NUM_SUBCORES + 1):
            b = sid + t * NUM_SUBCORES

            @pl.when(b < NZB)
            def _():
                pltpu.make_async_copy(rows_a, acc.at[pl.ds(b * ZB, ZB)],
                                      sem_a).wait()

        def make_gidx(pb, kb):
            def gidx(i, carry):
                for j in range(CHUNK // 16):
                    sl = pl.ds(j * 16, 16)
                    pb[i, sl] = kb[i, sl] * N + pb[i, sl]
                return carry
            return gidx

        pltpu.make_async_copy(er_hbm.at[0, wid, 0], p2a, sem_ma).wait()
        pltpu.make_async_copy(er_hbm.at[1, wid, 0], d2a, sem_ma).wait()
        pltpu.make_async_copy(kid_hbm.at[wid, 0], kid1, sem_k).wait()
        lax.fori_loop(0, BCH, make_gidx(p2a, kid1), 0)
        pltpu.make_async_copy(er_hbm.at[0, wid, 1], p2b, sem_mb).start()
        pltpu.make_async_copy(er_hbm.at[1, wid, 1], d2b, sem_mb).start()
        pltpu.make_async_copy(kid_hbm.at[wid, 1], kid1, sem_k).start()

        plsc.subcore_barrier()

        for bi in range(NBLK_E):
            if bi % 2 == 0:
                pb, db = p2a, d2a
                pn, dn, smn = p2b, d2b, sem_mb
            else:
                pb, db = p2b, d2b
                pn, dn, smn = p2a, d2a, sem_ma

            def gather(ci, rows, sem, pb=pb):
                return pltpu.make_async_copy(xw_hbm.at[pb.at[ci]], rows, sem)

            gather(0, rows_a, sem_a).start()
            gather(1, rows_b, sem_b).start()

            if bi + 1 < NBLK_E:
                pltpu.make_async_copy(er_hbm.at[0, wid, bi + 1], pn,
                                      smn).wait()
                pltpu.make_async_copy(er_hbm.at[1, wid, bi + 1], dn,
                                      smn).wait()
                pltpu.make_async_copy(kid_hbm.at[wid, bi + 1], kid1,
                                      sem_k).wait()
                lax.fori_loop(0, BCH, make_gidx(pn, kid1), 0)
                if bi + 2 < NBLK_E:
                    pltpu.make_async_copy(kid_hbm.at[wid, bi + 2], kid1,
                                          sem_k).start()

            def step(t, carry, gather=gather, db=db):
                c = 2 * t
                gather(c, rows_a, sem_a).wait()
                pltpu.sync_copy(rows_a, acc.at[db.at[c]], add=True)
                gather(c + 2, rows_a, sem_a).start()
                gather(c + 1, rows_b, sem_b).wait()
                pltpu.sync_copy(rows_b, acc.at[db.at[c + 1]], add=True)

                @pl.when(c + 3 < BCH)
                def _(gather=gather, c=c):
                    gather(c + 3, rows_b, sem_b).start()
                return carry
            lax.fori_loop(0, (BCH - 1) // 2, step, 0)

            gather(BCH - 1, rows_a, sem_a).wait()
            pltpu.sync_copy(rows_a, acc.at[db.at[BCH - 1]], add=True)

            if bi + 2 < NBLK_E:
                sm2 = sem_ma if bi % 2 == 0 else sem_mb
                pltpu.make_async_copy(er_hbm.at[0, wid, bi + 2], pb,
                                      sm2).start()
                pltpu.make_async_copy(er_hbm.at[1, wid, bi + 2], db,
                                      sm2).start()

        plsc.subcore_barrier()

        for t in range(NZB // NUM_SUBCORES + 1):
            b = sid + t * NUM_SUBCORES

            @pl.when(b < NZB)
            def _():
                pltpu.make_async_copy(acc.at[pl.ds(b * ZB, ZB)],
                                      out_hbm.at[cid, pl.ds(b * ZB, ZB)],
                                      sem_a).start()
        for t in range(NZB // NUM_SUBCORES + 1):
            b = sid + t * NUM_SUBCORES

            @pl.when(b < NZB)
            def _():
                pltpu.make_async_copy(acc.at[pl.ds(b * ZB, ZB)],
                                      out_hbm.at[cid, pl.ds(b * ZB, ZB)],
                                      sem_a).wait()

    return k(er, kidr, xw)


def _bn_relu(partial, gamma2, beta2):
    def body(p_ref, g_ref, b_ref, o_ref, s_vmem, st_vmem):
        p = pl.program_id(0)
        b = pl.program_id(1)

        @pl.when(p == 0)
        def _():
            s = p_ref[0] + p_ref[1]
            s_vmem[pl.ds(b * ST_ROWS, ST_ROWS), :] = s

            @pl.when(b == 0)
            def _():
                st_vmem[...] = jnp.zeros_like(st_vmem)

            st_vmem[0, :] += jnp.sum(s, axis=0)
            st_vmem[1, :] += jnp.sum(s * s, axis=0)
            o_ref[...] = s

        @pl.when(p == 1)
        def _():
            mean = st_vmem[0:1, :] * (1.0 / N)
            var = st_vmem[1:2, :] * (1.0 / N) - mean * mean
            scale = g_ref[...] * lax.rsqrt(var + EPS)
            s = s_vmem[pl.ds(b * ST_ROWS, ST_ROWS), :]
            o_ref[...] = jnp.maximum((s - mean) * scale + b_ref[...], 0.0)

    return pl.pallas_call(
        body,
        grid=(2, NBLK_ST),
        in_specs=[
            pl.BlockSpec((NUM_CORES, ST_ROWS, C),
                         lambda p, b: (0, b * (1 - p), 0)),
            pl.BlockSpec((1, C), lambda p, b: (0, 0)),
            pl.BlockSpec((1, C), lambda p, b: (0, 0)),
        ],
        out_specs=pl.BlockSpec((ST_ROWS, C), lambda p, b: (b, 0)),
        out_shape=jax.ShapeDtypeStruct((N, C), jnp.float32),
        scratch_shapes=[
            pltpu.VMEM((N, C), jnp.float32),
            pltpu.VMEM((8, C), jnp.float32),
        ],
    )(partial, gamma2, beta2)


def kernel(x, edge_index, kernel_id, W, gamma, beta):
    er = edge_index.astype(jnp.int32).reshape(2, NW, NBLK_E, BCH, CHUNK)
    kidr = kernel_id.astype(jnp.int32).reshape(NW, NBLK_E, BCH, CHUNK)

    xw = _xw_matmul(x, W)
    partial = _sc_edge_scatter(xw, er, kidr)
    return _bn_relu(partial, gamma.reshape(1, C), beta.reshape(1, C))


# speedup vs baseline: 1.1002x; 1.0949x over previous
import functools

import jax
import jax.numpy as jnp
from jax import lax
from jax.experimental import pallas as pl
from jax.experimental.pallas import tpu as pltpu
from jax.experimental.pallas import tpu_sc as plsc

N = 10000
E = 320000
C = 128
K = 27
EPS = 1e-5

NUM_CORES = 2
NUM_SUBCORES = 16
NW = NUM_CORES * NUM_SUBCORES
EPW = E // NW
CHUNK = 80
NCH = EPW // CHUNK
BCH = 25
NBLK_E = NCH // BCH
ZB = 80
NZB = N // ZB

BN_ROWS = 1000
NBLK = N // BN_ROWS
MM_ROWS = 5000
NBLK_MM = N // MM_ROWS
ST_ROWS = 2000
NBLK_ST = N // ST_ROWS


def _xw_matmul(x, W):
    def body(x_ref, w_ref, o_ref):
        o_ref[...] = jnp.dot(x_ref[...], w_ref[0],
                             preferred_element_type=jnp.float32
)

    return pl.pallas_call(
        body,
        grid=(NBLK_MM, K),
        in_specs=[
            pl.BlockSpec((MM_ROWS, C), lambda b, k: (b, 0)),
            pl.BlockSpec((1, C, C), lambda b, k: (k, 0, 0)),
        ],
        out_specs=pl.BlockSpec((MM_ROWS, C),
                               lambda b, k: (k * NBLK_MM + b, 0)),
        out_shape=jax.ShapeDtypeStruct((K * N, C), jnp.float32),
    )(x, W)


def _sc_edge_scatter(xw, er, kidr):
    mesh = plsc.VectorSubcoreMesh(core_axis_name="c", subcore_axis_name="s")

    @functools.partial(
        pl.kernel,
        mesh=mesh,
        out_type=jax.ShapeDtypeStruct((NUM_CORES, N, C), jnp.float32),
        scratch_types=[
            pltpu.VMEM((BCH, CHUNK), jnp.int32),
            pltpu.VMEM((BCH, CHUNK), jnp.int32),
            pltpu.VMEM((BCH, CHUNK), jnp.int32),
            pltpu.VMEM((BCH, CHUNK), jnp.int32),
            pltpu.VMEM((CHUNK, C), jnp.float32),
            pltpu.VMEM((CHUNK, C), jnp.float32),
            pltpu.VMEM((CHUNK, C), jnp.float32),
            pltpu.VMEM_SHARED((N, C), jnp.float32),
            pltpu.SemaphoreType.DMA,
            pltpu.SemaphoreType.DMA,
            pltpu.SemaphoreType.DMA,
            pltpu.SemaphoreType.DMA,
            pltpu.SemaphoreType.DMA,
        ],
    )
    def k(er_hbm, kid_hbm, xw_hbm, out_hbm,
          p2a, p2b, d2a, d2b, rows_a, rows_b, rows_c, acc,
          sem_a, sem_b, sem_c, sem_ma, sem_mb):
        cid = lax.axis_index("c")
        sid = lax.axis_index("s")
        wid = sid * NUM_CORES + cid

        pltpu.make_async_copy(er_hbm.at[0, wid, 0], p2a, sem_ma).start()
        pltpu.make_async_copy(kid_hbm.at[wid, 0], d2a, sem_ma).start()

        def zrow(i, carry):
            for j in range(C // 16):
                rows_a[i, pl.ds(j * 16, 16)] = jnp.zeros((16,), jnp.float32)
            return carry
        lax.fori_loop(0, CHUNK, zrow, 0)
        for t in range(NZB // NUM_SUBCORES + 1):
            b = sid + t * NUM_SUBCORES

            @pl.when(b < NZB)
            def _():
                pltpu.make_async_copy(rows_a, acc.at[pl.ds(b * ZB, ZB)],
                                      sem_a).start()
        for t in range(NZB // NUM_SUBCORES + 1):
            b = sid + t * NUM_SUBCORES

            @pl.when(b < NZB)
            def _():
                pltpu.make_async_copy(rows_a, acc.at[pl.ds(b * ZB, ZB)],
                                      sem_a).wait()

        def make_gidx(pb, kb):
            def gidx(i, carry):
                for j in range(CHUNK // 16):
                    sl = pl.ds(j * 16, 16)
                    pb[i, sl] = kb[i, sl] * N + pb[i, sl]
                return carry
            return gidx

        pltpu.make_async_copy(er_hbm.at[0, wid, 0], p2a, sem_ma).wait()
        pltpu.make_async_copy(kid_hbm.at[wid, 0], d2a, sem_ma).wait()
        lax.fori_loop(0, BCH, make_gidx(p2a, d2a), 0)
        pltpu.make_async_copy(er_hbm.at[1, wid, 0], d2a, sem_ma).start()
        pltpu.make_async_copy(er_hbm.at[0, wid, 1], p2b, sem_mb).start()
        pltpu.make_async_copy(kid_hbm.at[wid, 1], d2b, sem_mb).start()

        plsc.subcore_barrier()

        for bi in range(NBLK_E):
            if bi % 2 == 0:
                pb, db, sm = p2a, d2a, sem_ma
                pn, dn, smn = p2b, d2b, sem_mb
            else:
                pb, db, sm = p2b, d2b, sem_mb
                pn, dn, smn = p2a, d2a, sem_ma

            def gather(ci, rows, sem, pb=pb):
                return pltpu.make_async_copy(xw_hbm.at[pb.at[ci]], rows, sem)

            gather(0, rows_a, sem_a).start()
            gather(1, rows_b, sem_b).start()
            gather(2, rows_c, sem_c).start()

            if bi + 1 < NBLK_E:
                pltpu.make_async_copy(er_hbm.at[0, wid, bi + 1], pn,
                                      smn).wait()
                pltpu.make_async_copy(kid_hbm.at[wid, bi + 1], dn,
                                      smn).wait()
                lax.fori_loop(0, BCH, make_gidx(pn, dn), 0)
                pltpu.make_async_copy(er_hbm.at[1, wid, bi + 1], dn,
                                      smn).start()

            pltpu.make_async_copy(er_hbm.at[1, wid, bi], db, sm).wait()

            def step(t, carry, gather=gather, db=db):
                c = 3 * t
                for lane, (rw, sm_) in enumerate([(rows_a, sem_a),
                                                  (rows_b, sem_b),
                                                  (rows_c, sem_c)]):
                    gather(c + lane, rw, sm_).wait()
                    pltpu.sync_copy(rw, acc.at[db.at[c + lane]], add=True)

                    @pl.when(c + lane + 3 < BCH)
                    def _(gather=gather, c=c, lane=lane, rw=rw, sm_=sm_):
                        gather(c + lane + 3, rw, sm_).start()
                return carry
            lax.fori_loop(0, (BCH - 1) // 3, step, 0)

            gather(BCH - 1, rows_a, sem_a).wait()
            pltpu.sync_copy(rows_a, acc.at[db.at[BCH - 1]], add=True)

            if bi + 2 < NBLK_E:
                pltpu.make_async_copy(er_hbm.at[0, wid, bi + 2], pb,
                                      sm).start()
                pltpu.make_async_copy(kid_hbm.at[wid, bi + 2], db,
                                      sm).start()

        plsc.subcore_barrier()

        for t in range(NZB // NUM_SUBCORES + 1):
            b = sid + t * NUM_SUBCORES

            @pl.when(b < NZB)
            def _():
                pltpu.make_async_copy(acc.at[pl.ds(b * ZB, ZB)],
                                      out_hbm.at[cid, pl.ds(b * ZB, ZB)],
                                      sem_a).start()
        for t in range(NZB // NUM_SUBCORES + 1):
            b = sid + t * NUM_SUBCORES

            @pl.when(b < NZB)
            def _():
                pltpu.make_async_copy(acc.at[pl.ds(b * ZB, ZB)],
                                      out_hbm.at[cid, pl.ds(b * ZB, ZB)],
                                      sem_a).wait()

    return k(er, kidr, xw)


def _bn_relu(partial, gamma2, beta2):
    def body(p_ref, g_ref, b_ref, o_ref, s_vmem, st_vmem):
        p = pl.program_id(0)
        b = pl.program_id(1)

        @pl.when(p == 0)
        def _():
            s = p_ref[0] + p_ref[1]
            s_vmem[pl.ds(b * ST_ROWS, ST_ROWS), :] = s

            @pl.when(b == 0)
            def _():
                st_vmem[...] = jnp.zeros_like(st_vmem)

            st_vmem[0, :] += jnp.sum(s, axis=0)
            st_vmem[1, :] += jnp.sum(s * s, axis=0)
            o_ref[...] = s

        @pl.when(p == 1)
        def _():
            mean = st_vmem[0:1, :] * (1.0 / N)
            var = st_vmem[1:2, :] * (1.0 / N) - mean * mean
            scale = g_ref[...] * lax.rsqrt(var + EPS)
            s = s_vmem[pl.ds(b * ST_ROWS, ST_ROWS), :]
            o_ref[...] = jnp.maximum((s - mean) * scale + b_ref[...], 0.0)

    return pl.pallas_call(
        body,
        grid=(2, NBLK_ST),
        in_specs=[
            pl.BlockSpec((NUM_CORES, ST_ROWS, C),
                         lambda p, b: (0, b * (1 - p), 0)),
            pl.BlockSpec((1, C), lambda p, b: (0, 0)),
            pl.BlockSpec((1, C), lambda p, b: (0, 0)),
        ],
        out_specs=pl.BlockSpec((ST_ROWS, C), lambda p, b: (b, 0)),
        out_shape=jax.ShapeDtypeStruct((N, C), jnp.float32),
        scratch_shapes=[
            pltpu.VMEM((N, C), jnp.float32),
            pltpu.VMEM((8, C), jnp.float32),
        ],
    )(partial, gamma2, beta2)


def kernel(x, edge_index, kernel_id, W, gamma, beta):
    er = edge_index.astype(jnp.int32).reshape(2, NW, NBLK_E, BCH, CHUNK)
    kidr = kernel_id.astype(jnp.int32).reshape(NW, NBLK_E, BCH, CHUNK)

    xw = _xw_matmul(x, W)
    partial = _sc_edge_scatter(xw, er, kidr)
    return _bn_relu(partial, gamma.reshape(1, C), beta.reshape(1, C))


# speedup vs baseline: 1.2389x; 1.1260x over previous
import functools

import jax
import jax.numpy as jnp
from jax import lax
from jax.experimental import pallas as pl
from jax.experimental.pallas import tpu as pltpu
from jax.experimental.pallas import tpu_sc as plsc

N = 10000
E = 320000
C = 128
K = 27
EPS = 1e-5

NUM_CORES = 2
NUM_SUBCORES = 16
NW = NUM_CORES * NUM_SUBCORES
EPW = E // NW
CHUNK = 80
NCH = EPW // CHUNK
BCH = 25
NBLK_E = NCH // BCH
ZB = 80
NZB = N // ZB

BN_ROWS = 1000
NBLK = N // BN_ROWS
MM_ROWS = 10000
NBLK_MM = N // MM_ROWS
ST_ROWS = 2000
NBLK_ST = N // ST_ROWS


def _xw_matmul(x, W):
    def body(x_ref, w_ref, o_ref):
        o_ref[...] = jnp.dot(x_ref[...], w_ref[0],
                             preferred_element_type=jnp.float32
)

    return pl.pallas_call(
        body,
        grid=(NBLK_MM, K),
        in_specs=[
            pl.BlockSpec((MM_ROWS, C), lambda b, k: (b, 0)),
            pl.BlockSpec((1, C, C), lambda b, k: (k, 0, 0)),
        ],
        out_specs=pl.BlockSpec((MM_ROWS, C),
                               lambda b, k: (k * NBLK_MM + b, 0)),
        out_shape=jax.ShapeDtypeStruct((K * N, C), jnp.float32),
    )(x, W)


def _sc_edge_scatter(xw, er, kidr):
    mesh = plsc.VectorSubcoreMesh(core_axis_name="c", subcore_axis_name="s")

    @functools.partial(
        pl.kernel,
        mesh=mesh,
        out_type=jax.ShapeDtypeStruct((NUM_CORES, N, C), jnp.float32),
        scratch_types=[
            pltpu.VMEM((BCH, CHUNK), jnp.int32),
            pltpu.VMEM((BCH, CHUNK), jnp.int32),
            pltpu.VMEM((BCH, CHUNK), jnp.int32),
            pltpu.VMEM((BCH, CHUNK), jnp.int32),
            pltpu.VMEM((CHUNK, C), jnp.float32),
            pltpu.VMEM((CHUNK, C), jnp.float32),
            pltpu.VMEM((CHUNK, C), jnp.float32),
            pltpu.VMEM_SHARED((N, C), jnp.float32),
            pltpu.SemaphoreType.DMA,
            pltpu.SemaphoreType.DMA,
            pltpu.SemaphoreType.DMA,
            pltpu.SemaphoreType.DMA,
            pltpu.SemaphoreType.DMA,
        ],
    )
    def k(er_hbm, kid_hbm, xw_hbm, out_hbm,
          p2a, p2b, d2a, d2b, rows_a, rows_b, rows_c, acc,
          sem_a, sem_b, sem_c, sem_ma, sem_mb):
        cid = lax.axis_index("c")
        sid = lax.axis_index("s")
        wid = sid * NUM_CORES + cid

        pltpu.make_async_copy(er_hbm.at[0, wid, 0], p2a, sem_ma).start()
        pltpu.make_async_copy(kid_hbm.at[wid, 0], d2a, sem_ma).start()

        def zrow(i, carry):
            for j in range(C // 16):
                rows_a[i, pl.ds(j * 16, 16)] = jnp.zeros((16,), jnp.float32)
            return carry
        lax.fori_loop(0, CHUNK, zrow, 0)
        for t in range(NZB // NUM_SUBCORES + 1):
            b = sid + t * NUM_SUBCORES

            @pl.when(b < NZB)
            def _():
                pltpu.make_async_copy(rows_a, acc.at[pl.ds(b * ZB, ZB)],
                                      sem_a).start()
        for t in range(NZB // NUM_SUBCORES + 1):
            b = sid + t * NUM_SUBCORES

            @pl.when(b < NZB)
            def _():
                pltpu.make_async_copy(rows_a, acc.at[pl.ds(b * ZB, ZB)],
                                      sem_a).wait()

        def make_gidx(pb, kb):
            def gidx(i, carry):
                for j in range(CHUNK // 16):
                    sl = pl.ds(j * 16, 16)
                    pb[i, sl] = kb[i, sl] * N + pb[i, sl]
                return carry
            return gidx

        pltpu.make_async_copy(er_hbm.at[0, wid, 0], p2a, sem_ma).wait()
        pltpu.make_async_copy(kid_hbm.at[wid, 0], d2a, sem_ma).wait()
        lax.fori_loop(0, BCH, make_gidx(p2a, d2a), 0)
        pltpu.make_async_copy(er_hbm.at[1, wid, 0], d2a, sem_ma).start()
        pltpu.make_async_copy(er_hbm.at[0, wid, 1], p2b, sem_mb).start()
        pltpu.make_async_copy(kid_hbm.at[wid, 1], d2b, sem_mb).start()

        plsc.subcore_barrier()

        for bi in range(NBLK_E):
            if bi % 2 == 0:
                pb, db, sm = p2a, d2a, sem_ma
                pn, dn, smn = p2b, d2b, sem_mb
            else:
                pb, db, sm = p2b, d2b, sem_mb
                pn, dn, smn = p2a, d2a, sem_ma

            def gather(ci, rows, sem, pb=pb):
                return pltpu.make_async_copy(xw_hbm.at[pb.at[ci]], rows, sem)

            gather(0, rows_a, sem_a).start()
            gather(1, rows_b, sem_b).start()
            gather(2, rows_c, sem_c).start()

            if bi + 1 < NBLK_E:
                pltpu.make_async_copy(er_hbm.at[0, wid, bi + 1], pn,
                                      smn).wait()
                pltpu.make_async_copy(kid_hbm.at[wid, bi + 1], dn,
                                      smn).wait()
                lax.fori_loop(0, BCH, make_gidx(pn, dn), 0)
                pltpu.make_async_copy(er_hbm.at[1, wid, bi + 1], dn,
                                      smn).start()

            pltpu.make_async_copy(er_hbm.at[1, wid, bi], db, sm).wait()

            def step(t, carry, gather=gather, db=db):
                c = 3 * t
                for lane, (rw, sm_) in enumerate([(rows_a, sem_a),
                                                  (rows_b, sem_b),
                                                  (rows_c, sem_c)]):
                    gather(c + lane, rw, sm_).wait()
                    pltpu.sync_copy(rw, acc.at[db.at[c + lane]], add=True)

                    @pl.when(c + lane + 3 < BCH)
                    def _(gather=gather, c=c, lane=lane, rw=rw, sm_=sm_):
                        gather(c + lane + 3, rw, sm_).start()
                return carry
            lax.fori_loop(0, (BCH - 1) // 3, step, 0)

            gather(BCH - 1, rows_a, sem_a).wait()
            pltpu.sync_copy(rows_a, acc.at[db.at[BCH - 1]], add=True)

            if bi + 2 < NBLK_E:
                pltpu.make_async_copy(er_hbm.at[0, wid, bi + 2], pb,
                                      sm).start()
                pltpu.make_async_copy(kid_hbm.at[wid, bi + 2], db,
                                      sm).start()

        plsc.subcore_barrier()

        for t in range(NZB // NUM_SUBCORES + 1):
            b = sid + t * NUM_SUBCORES

            @pl.when(b < NZB)
            def _():
                pltpu.make_async_copy(acc.at[pl.ds(b * ZB, ZB)],
                                      out_hbm.at[cid, pl.ds(b * ZB, ZB)],
                                      sem_a).start()
        for t in range(NZB // NUM_SUBCORES + 1):
            b = sid + t * NUM_SUBCORES

            @pl.when(b < NZB)
            def _():
                pltpu.make_async_copy(acc.at[pl.ds(b * ZB, ZB)],
                                      out_hbm.at[cid, pl.ds(b * ZB, ZB)],
                                      sem_a).wait()

    return k(er, kidr, xw)


def _bn_relu(partial, gamma2, beta2):
    def body(p_ref, g_ref, b_ref, o_ref, s_vmem, st_vmem):
        p = pl.program_id(0)
        b = pl.program_id(1)

        @pl.when(p == 0)
        def _():
            s = p_ref[0] + p_ref[1]
            s_vmem[pl.ds(b * ST_ROWS, ST_ROWS), :] = s

            @pl.when(b == 0)
            def _():
                st_vmem[...] = jnp.zeros_like(st_vmem)

            st_vmem[0, :] += jnp.sum(s, axis=0)
            st_vmem[1, :] += jnp.sum(s * s, axis=0)
            o_ref[...] = s

        @pl.when(p == 1)
        def _():
            mean = st_vmem[0:1, :] * (1.0 / N)
            var = st_vmem[1:2, :] * (1.0 / N) - mean * mean
            scale = g_ref[...] * lax.rsqrt(var + EPS)
            s = s_vmem[pl.ds(b * ST_ROWS, ST_ROWS), :]
            o_ref[...] = jnp.maximum((s - mean) * scale + b_ref[...], 0.0)

    return pl.pallas_call(
        body,
        grid=(2, NBLK_ST),
        in_specs=[
            pl.BlockSpec((NUM_CORES, ST_ROWS, C),
                         lambda p, b: (0, b * (1 - p), 0)),
            pl.BlockSpec((1, C), lambda p, b: (0, 0)),
            pl.BlockSpec((1, C), lambda p, b: (0, 0)),
        ],
        out_specs=pl.BlockSpec((ST_ROWS, C), lambda p, b: (b, 0)),
        out_shape=jax.ShapeDtypeStruct((N, C), jnp.float32),
        scratch_shapes=[
            pltpu.VMEM((N, C), jnp.float32),
            pltpu.VMEM((8, C), jnp.float32),
        ],
    )(partial, gamma2, beta2)


def kernel(x, edge_index, kernel_id, W, gamma, beta):
    er = edge_index.astype(jnp.int32).reshape(2, NW, NBLK_E, BCH, CHUNK)
    kidr = kernel_id.astype(jnp.int32).reshape(NW, NBLK_E, BCH, CHUNK)

    xw = _xw_matmul(x, W)
    partial = _sc_edge_scatter(xw, er, kidr)
    return _bn_relu(partial, gamma.reshape(1, C), beta.reshape(1, C))


# speedup vs baseline: 1.2396x; 1.0006x over previous
import functools

import jax
import jax.numpy as jnp
from jax import lax
from jax.experimental import pallas as pl
from jax.experimental.pallas import tpu as pltpu
from jax.experimental.pallas import tpu_sc as plsc

N = 10000
E = 320000
C = 128
K = 27
EPS = 1e-5

NUM_CORES = 2
NUM_SUBCORES = 16
NW = NUM_CORES * NUM_SUBCORES
EPW = E // NW
CHUNK = 80
NCH = EPW // CHUNK
BCH = 25
NBLK_E = NCH // BCH
ZB = 80
NZB = N // ZB

MM_ROWS = 10000
NBLK_MM = N // MM_ROWS
ST_ROWS = 2000
NBLK_ST = N // ST_ROWS


def _xw_matmul(x, W):
    def body(x_ref, w_ref, o_ref):
        o_ref[...] = jnp.dot(x_ref[...], w_ref[0],
                             preferred_element_type=jnp.float32
)

    return pl.pallas_call(
        body,
        grid=(NBLK_MM, K),
        in_specs=[
            pl.BlockSpec((MM_ROWS, C), lambda b, k: (b, 0)),
            pl.BlockSpec((1, C, C), lambda b, k: (k, 0, 0)),
        ],
        out_specs=pl.BlockSpec((MM_ROWS, C),
                               lambda b, k: (k * NBLK_MM + b, 0)),
        out_shape=jax.ShapeDtypeStruct((K * N, C), jnp.float32),
    )(x, W)


def _sc_edge_scatter(xw, er, kidr):
    mesh = plsc.VectorSubcoreMesh(core_axis_name="c", subcore_axis_name="s")

    @functools.partial(
        pl.kernel,
        mesh=mesh,
        out_type=jax.ShapeDtypeStruct((NUM_CORES, N, C), jnp.float32),
        scratch_types=[
            pltpu.VMEM((BCH, CHUNK), jnp.int32),
            pltpu.VMEM((BCH, CHUNK), jnp.int32),
            pltpu.VMEM((BCH, CHUNK), jnp.int32),
            pltpu.VMEM((BCH, CHUNK), jnp.int32),
            pltpu.VMEM((CHUNK, C), jnp.float32),
            pltpu.VMEM((CHUNK, C), jnp.float32),
            pltpu.VMEM((CHUNK, C), jnp.float32),
            pltpu.VMEM_SHARED((N, C), jnp.float32),
            pltpu.SemaphoreType.DMA,
            pltpu.SemaphoreType.DMA,
            pltpu.SemaphoreType.DMA,
            pltpu.SemaphoreType.DMA,
            pltpu.SemaphoreType.DMA,
        ],
    )
    def k(er_hbm, kid_hbm, xw_hbm, out_hbm,
          p2a, p2b, d2a, d2b, rows_a, rows_b, rows_c, acc,
          sem_a, sem_b, sem_c, sem_ma, sem_mb):
        cid = lax.axis_index("c")
        sid = lax.axis_index("s")
        wid = sid * NUM_CORES + cid

        pltpu.make_async_copy(er_hbm.at[0, wid, 0], p2a, sem_ma).start()
        pltpu.make_async_copy(kid_hbm.at[wid, 0], d2a, sem_ma).start()

        def zrow(i, carry):
            for j in range(C // 16):
                rows_a[i, pl.ds(j * 16, 16)] = jnp.zeros((16,), jnp.float32)
            return carry
        lax.fori_loop(0, CHUNK, zrow, 0)
        for t in range(NZB // NUM_SUBCORES + 1):
            b = sid + t * NUM_SUBCORES

            @pl.when(b < NZB)
            def _():
                pltpu.make_async_copy(rows_a, acc.at[pl.ds(b * ZB, ZB)],
                                      sem_a).start()
        for t in range(NZB // NUM_SUBCORES + 1):
            b = sid + t * NUM_SUBCORES

            @pl.when(b < NZB)
            def _():
                pltpu.make_async_copy(rows_a, acc.at[pl.ds(b * ZB, ZB)],
                                      sem_a).wait()

        def make_gidx(pb, kb):
            def gidx(i, carry):
                for j in range(CHUNK // 16):
                    sl = pl.ds(j * 16, 16)
                    pb[i, sl] = kb[i, sl] * N + pb[i, sl]
                return carry
            return gidx

        pltpu.make_async_copy(er_hbm.at[0, wid, 0], p2a, sem_ma).wait()
        pltpu.make_async_copy(kid_hbm.at[wid, 0], d2a, sem_ma).wait()
        lax.fori_loop(0, BCH, make_gidx(p2a, d2a), 0)
        pltpu.make_async_copy(er_hbm.at[1, wid, 0], d2a, sem_ma).start()
        pltpu.make_async_copy(er_hbm.at[0, wid, 1], p2b, sem_mb).start()
        pltpu.make_async_copy(kid_hbm.at[wid, 1], d2b, sem_mb).start()

        plsc.subcore_barrier()

        for bi in range(NBLK_E):
            if bi % 2 == 0:
                pb, db, sm = p2a, d2a, sem_ma
                pn, dn, smn = p2b, d2b, sem_mb
            else:
                pb, db, sm = p2b, d2b, sem_mb
                pn, dn, smn = p2a, d2a, sem_ma

            def gather(ci, rows, sem, pb=pb):
                return pltpu.make_async_copy(xw_hbm.at[pb.at[ci]], rows, sem)

            gather(0, rows_a, sem_a).start()
            gather(1, rows_b, sem_b).start()
            gather(2, rows_c, sem_c).start()

            if bi + 1 < NBLK_E:
                pltpu.make_async_copy(er_hbm.at[0, wid, bi + 1], pn,
                                      smn).wait()
                pltpu.make_async_copy(kid_hbm.at[wid, bi + 1], dn,
                                      smn).wait()
                lax.fori_loop(0, BCH, make_gidx(pn, dn), 0)
                pltpu.make_async_copy(er_hbm.at[1, wid, bi + 1], dn,
                                      smn).start()

            pltpu.make_async_copy(er_hbm.at[1, wid, bi], db, sm).wait()

            def step(t, carry, gather=gather, db=db):
                c = 3 * t
                for lane, (rw, sm_) in enumerate([(rows_a, sem_a),
                                                  (rows_b, sem_b),
                                                  (rows_c, sem_c)]):
                    gather(c + lane, rw, sm_).wait()
                    pltpu.sync_copy(rw, acc.at[db.at[c + lane]], add=True)

                    @pl.when(c + lane + 3 < BCH)
                    def _(gather=gather, c=c, lane=lane, rw=rw, sm_=sm_):
                        gather(c + lane + 3, rw, sm_).start()
                return carry
            lax.fori_loop(0, (BCH - 1) // 3, step, 0)

            gather(BCH - 1, rows_a, sem_a).wait()
            pltpu.sync_copy(rows_a, acc.at[db.at[BCH - 1]], add=True)

            if bi + 2 < NBLK_E:
                pltpu.make_async_copy(er_hbm.at[0, wid, bi + 2], pb,
                                      sm).start()
                pltpu.make_async_copy(kid_hbm.at[wid, bi + 2], db,
                                      sm).start()

        plsc.subcore_barrier()

        for t in range(NZB // NUM_SUBCORES + 1):
            b = sid + t * NUM_SUBCORES

            @pl.when(b < NZB)
            def _():
                pltpu.make_async_copy(acc.at[pl.ds(b * ZB, ZB)],
                                      out_hbm.at[cid, pl.ds(b * ZB, ZB)],
                                      sem_a).start()
        for t in range(NZB // NUM_SUBCORES + 1):
            b = sid + t * NUM_SUBCORES

            @pl.when(b < NZB)
            def _():
                pltpu.make_async_copy(acc.at[pl.ds(b * ZB, ZB)],
                                      out_hbm.at[cid, pl.ds(b * ZB, ZB)],
                                      sem_a).wait()

    return k(er, kidr, xw)


def _bn_relu(partial, gamma2, beta2):
    def body(p_ref, g_ref, b_ref, o_ref, s_vmem, st_vmem):
        p = pl.program_id(0)
        b = pl.program_id(1)

        @pl.when(p == 0)
        def _():
            s = p_ref[0] + p_ref[1]
            s_vmem[pl.ds(b * ST_ROWS, ST_ROWS), :] = s

            @pl.when(b == 0)
            def _():
                st_vmem[...] = jnp.zeros_like(st_vmem)

            st_vmem[0, :] += jnp.sum(s, axis=0)
            st_vmem[1, :] += jnp.sum(s * s, axis=0)
            o_ref[...] = s

        @pl.when(p == 1)
        def _():
            mean = st_vmem[0:1, :] * (1.0 / N)
            var = st_vmem[1:2, :] * (1.0 / N) - mean * mean
            scale = g_ref[...] * lax.rsqrt(var + EPS)
            s = s_vmem[pl.ds(b * ST_ROWS, ST_ROWS), :]
            o_ref[...] = jnp.maximum((s - mean) * scale + b_ref[...], 0.0)

    return pl.pallas_call(
        body,
        grid=(2, NBLK_ST),
        in_specs=[
            pl.BlockSpec((NUM_CORES, ST_ROWS, C),
                         lambda p, b: (0, b * (1 - p), 0)),
            pl.BlockSpec((1, C), lambda p, b: (0, 0)),
            pl.BlockSpec((1, C), lambda p, b: (0, 0)),
        ],
        out_specs=pl.BlockSpec((ST_ROWS, C), lambda p, b: (b, 0)),
        out_shape=jax.ShapeDtypeStruct((N, C), jnp.float32),
        scratch_shapes=[
            pltpu.VMEM((N, C), jnp.float32),
            pltpu.VMEM((8, C), jnp.float32),
        ],
    )(partial, gamma2, beta2)


def kernel(x, edge_index, kernel_id, W, gamma, beta):
    er = edge_index.astype(jnp.int32).reshape(2, NW, NBLK_E, BCH, CHUNK)
    kidr = kernel_id.astype(jnp.int32).reshape(NW, NBLK_E, BCH, CHUNK)

    xw = _xw_matmul(x, W)
    partial = _sc_edge_scatter(xw, er, kidr)
    return _bn_relu(partial, gamma.reshape(1, C), beta.reshape(1, C))
